# baseline (speedup 1.0000x reference)
import functools

import jax
import jax.numpy as jnp
from jax import lax
from jax.experimental import pallas as pl
from jax.experimental.pallas import tpu as pltpu
from jax.experimental.pallas import tpu_sc as plsc

NC = 2
NS = 16
L = 16
NW = NC * NS

B = 16
A = 262144
TOTAL = B * A
GROUP = 128
NGROUPS = TOTAL // GROUP

SC_FRAC16 = 10
SC_ANCHORS = TOTAL * SC_FRAC16 // 16
SC_GROUPS = SC_ANCHORS // GROUP

PER_W = SC_ANCHORS // NW
CHUNK = 8192
NCHUNK = PER_W // CHUNK
UNROLL = 16
STEPS = CHUNK // (UNROLL * L)

_LOG1P_C = (
    -0.05486231128935009,
    0.2164085836818178,
    -0.46407070110262433,
    0.9954266617754363,
    0.00014158017492720682,
)


def _sc_body(match_hbm, logits_hbm, out_sum, out_cnt,
             mb0, mb1, lb0, lb1, acc_s, cnt_s, sm0, sm1, sl0, sl1):
    cid = lax.axis_index("c")
    sid = lax.axis_index("s")
    wid = sid * NC + cid
    mbase = wid * PER_W

    mbufs = (mb0, mb1)
    lbufs = (lb0, lb1)
    msems = (sm0, sm1)
    lsems = (sl0, sl1)

    def start(k):
        slot = k % 2
        off = mbase + k * CHUNK
        cm = pltpu.async_copy(match_hbm.at[pl.ds(off, CHUNK)],
                              mbufs[slot], msems[slot])
        cl = pltpu.async_copy(logits_hbm.at[pl.ds(2 * off, 2 * CHUNK)],
                              lbufs[slot], lsems[slot])
        return cm, cl

    pending = start(0)
    acc = jnp.zeros((L,), jnp.float32)
    accd = jnp.zeros((L,), jnp.float32)
    cnt = jnp.zeros((L,), jnp.float32)
    for k in range(NCHUNK):
        nxt = start(k + 1) if k + 1 < NCHUNK else None
        pending[0].wait()
        pending[1].wait()
        mb = mbufs[k % 2]
        lb = lbufs[k % 2]

        def body(j, carry, mb=mb, lb=lb):
            acc, accd, cnt = carry
            base = j * (UNROLL * L)
            for u in range(UNROLL):
                mo = base + u * L
                g, r = divmod(mo, GROUP)
                lo = 2 * GROUP * g + r
                m = mb[pl.ds(mo, L)]
                l0 = lb[pl.ds(lo, L)]
                l1 = lb[pl.ds(lo + GROUP, L)]
                d = l1 - l0
                md = m * d
                w2 = m * m
                a = jnp.abs(d)
                e = jnp.exp(-a)
                p = jnp.full((L,), _LOG1P_C[0], jnp.float32)
                for c in _LOG1P_C[1:]:
                    p = p * e + c
                acc = acc + w2 * (p + 0.5 * a)
                accd = accd + md
                cnt = cnt + w2
            return acc, accd, cnt

        acc, accd, cnt = lax.fori_loop(0, STEPS, body, (acc, accd, cnt))
        pending = nxt

    acc_s[...] = acc - 0.5 * accd
    cnt_s[...] = cnt
    pltpu.sync_copy(acc_s, out_sum.at[wid])
    pltpu.sync_copy(cnt_s, out_cnt.at[wid])


_rpn_loss_sc = functools.partial(
    pl.kernel,
    out_type=(jax.ShapeDtypeStruct((NW, L), jnp.float32),
              jax.ShapeDtypeStruct((NW, L), jnp.float32)),
    mesh=plsc.VectorSubcoreMesh(core_axis_name="c", subcore_axis_name="s",
                                num_cores=NC, num_subcores=NS),
    compiler_params=pltpu.CompilerParams(needs_layout_passes=False),
    scratch_types=[
        pltpu.VMEM((CHUNK,), jnp.float32),
        pltpu.VMEM((CHUNK,), jnp.float32),
        pltpu.VMEM((2 * CHUNK,), jnp.float32),
        pltpu.VMEM((2 * CHUNK,), jnp.float32),
        pltpu.VMEM((L,), jnp.float32),
        pltpu.VMEM((L,), jnp.float32),
        pltpu.SemaphoreType.DMA,
        pltpu.SemaphoreType.DMA,
        pltpu.SemaphoreType.DMA,
        pltpu.SemaphoreType.DMA,
    ],
)(_sc_body)


TCG = 512
TC_GROUPS = NGROUPS - SC_GROUPS
TC_STEPS = TC_GROUPS // TCG
TC_BLK0 = SC_GROUPS // TCG


def _tc_body(m_ref, x_ref, sum_ref, cnt_ref):
    i = pl.program_id(0)

    @pl.when(i == 0)
    def _init():
        sum_ref[...] = jnp.zeros_like(sum_ref)
        cnt_ref[...] = jnp.zeros_like(cnt_ref)

    m = m_ref[...]
    x = x_ref[...]
    x4 = x.reshape(TCG, 2, 128)
    xt = jnp.transpose(x4, (1, 0, 2))
    l0 = xt[0]
    l1 = xt[1]
    d = l1 - l0
    md = m * d
    w2 = m * m
    a = jnp.abs(d)
    e = jnp.exp2(a * jnp.float32(-1.4426950408889634))
    p = jnp.full(e.shape, _LOG1P_C[0], jnp.float32)
    for c in _LOG1P_C[1:]:
        p = p * e + c
    ce = p - jnp.minimum(md, 0.0)
    sum_ref[...] += w2 * ce
    cnt_ref[...] += w2


def _rpn_loss_tc(match2d, logits2d):
    return pl.pallas_call(
        _tc_body,
        grid=(TC_STEPS,),
        in_specs=[
            pl.BlockSpec((TCG, 128), lambda i: (TC_BLK0 + i, 0)),
            pl.BlockSpec((2 * TCG, 128), lambda i: (TC_BLK0 + i, 0)),
        ],
        out_specs=[
            pl.BlockSpec((TCG, 128), lambda i: (0, 0)),
            pl.BlockSpec((TCG, 128), lambda i: (0, 0)),
        ],
        out_shape=[jax.ShapeDtypeStruct((TCG, 128), jnp.float32)] * 2,
    )(match2d, logits2d)


def kernel(rpn_match, rpn_class_logits):
    m_flat = rpn_match.reshape(TOTAL)
    lg_flat = (rpn_class_logits
               .reshape(B, A // GROUP, GROUP, 2)
               .transpose(0, 1, 3, 2)
               .reshape(TOTAL * 2))
    sc_sum, sc_cnt = _rpn_loss_sc(m_flat, lg_flat)
    tc_sum, tc_cnt = _rpn_loss_tc(m_flat.reshape(NGROUPS, GROUP),
                                  lg_flat.reshape(2 * NGROUPS, GROUP))
    s = jnp.sum(sc_sum) + jnp.sum(tc_sum)
    c = jnp.sum(sc_cnt) + jnp.sum(tc_cnt)
    return jnp.where(c > 0, s / jnp.maximum(c, 1.0), jnp.float32(0.0))


# speedup vs baseline: 1.1320x; 1.1320x over previous
import functools

import jax
import jax.numpy as jnp
from jax import lax
from jax.experimental import pallas as pl
from jax.experimental.pallas import tpu as pltpu
from jax.experimental.pallas import tpu_sc as plsc

NC = 2
NS = 16
L = 16
NW = NC * NS

B = 16
A = 262144
TOTAL = B * A
GROUP = 128
NGROUPS = TOTAL // GROUP

SC_FRAC16 = 10
SC_ANCHORS = TOTAL * SC_FRAC16 // 16
SC_GROUPS = SC_ANCHORS // GROUP

PER_W = SC_ANCHORS // NW
CHUNK = 8192
NCHUNK = PER_W // CHUNK
STEPS = CHUNK // GROUP
UNROLL = GROUP // L

_LOG1P_C = (
    -0.05486231128935009,
    0.2164085836818178,
    -0.46407070110262433,
    0.9954266617754363,
    0.00014158017492720682,
)


def _sc_body(match_hbm, logits_hbm, out_sum, out_cnt,
             mb0, mb1, lb0, lb1, acc_s, cnt_s, sm0, sm1, sl0, sl1):
    cid = lax.axis_index("c")
    sid = lax.axis_index("s")
    wid = sid * NC + cid
    mbase = wid * PER_W

    mbufs = (mb0, mb1)
    lbufs = (lb0, lb1)
    msems = (sm0, sm1)
    lsems = (sl0, sl1)

    def start(k):
        slot = k % 2
        off = mbase + k * CHUNK
        cm = pltpu.async_copy(match_hbm.at[pl.ds(off, CHUNK)],
                              mbufs[slot], msems[slot])
        cl = pltpu.async_copy(logits_hbm.at[pl.ds(2 * off, 2 * CHUNK)],
                              lbufs[slot], lsems[slot])
        return cm, cl

    pending = start(0)
    acc = jnp.zeros((L,), jnp.float32)
    cnt = jnp.zeros((L,), jnp.float32)
    for k in range(NCHUNK):
        nxt = start(k + 1) if k + 1 < NCHUNK else None
        pending[0].wait()
        pending[1].wait()
        mb = mbufs[k % 2]
        lb = lbufs[k % 2]

        def body(j, carry, mb=mb, lb=lb):
            acc, cnt = carry
            mo = j * GROUP
            lo = j * (2 * GROUP)
            for u in range(UNROLL):
                m = mb[pl.ds(mo + u * L, L)]
                l0 = lb[pl.ds(lo + u * L, L)]
                l1 = lb[pl.ds(lo + GROUP + u * L, L)]
                d = l1 - l0
                md = m * d
                w2 = m * m
                a = jnp.abs(d)
                e = jnp.exp(-a)
                p = jnp.full((L,), _LOG1P_C[0], jnp.float32)
                for c in _LOG1P_C[1:]:
                    p = p * e + c
                ce = p - jnp.minimum(md, 0.0)
                acc = acc + w2 * ce
                cnt = cnt + w2
            return acc, cnt

        acc, cnt = lax.fori_loop(0, STEPS, body, (acc, cnt))
        pending = nxt

    acc_s[...] = acc
    cnt_s[...] = cnt
    pltpu.sync_copy(acc_s, out_sum.at[wid])
    pltpu.sync_copy(cnt_s, out_cnt.at[wid])


_rpn_loss_sc = functools.partial(
    pl.kernel,
    out_type=(jax.ShapeDtypeStruct((NW, L), jnp.float32),
              jax.ShapeDtypeStruct((NW, L), jnp.float32)),
    mesh=plsc.VectorSubcoreMesh(core_axis_name="c", subcore_axis_name="s",
                                num_cores=NC, num_subcores=NS),
    compiler_params=pltpu.CompilerParams(needs_layout_passes=False),
    scratch_types=[
        pltpu.VMEM((CHUNK,), jnp.float32),
        pltpu.VMEM((CHUNK,), jnp.float32),
        pltpu.VMEM((2 * CHUNK,), jnp.float32),
        pltpu.VMEM((2 * CHUNK,), jnp.float32),
        pltpu.VMEM((L,), jnp.float32),
        pltpu.VMEM((L,), jnp.float32),
        pltpu.SemaphoreType.DMA,
        pltpu.SemaphoreType.DMA,
        pltpu.SemaphoreType.DMA,
        pltpu.SemaphoreType.DMA,
    ],
)(_sc_body)


TCG = 512
TC_GROUPS = NGROUPS - SC_GROUPS
TC_STEPS = TC_GROUPS // TCG
TC_BLK0 = SC_GROUPS // TCG


def _tc_body(m_ref, x_ref, sum_ref, cnt_ref):
    i = pl.program_id(0)

    @pl.when(i == 0)
    def _init():
        sum_ref[...] = jnp.zeros_like(sum_ref)
        cnt_ref[...] = jnp.zeros_like(cnt_ref)

    m = m_ref[...]
    x = x_ref[...]
    x4 = x.reshape(TCG, 2, 128)
    xt = jnp.transpose(x4, (1, 0, 2))
    l0 = xt[0]
    l1 = xt[1]
    d = l1 - l0
    md = m * d
    w2 = m * m
    a = jnp.abs(d)
    e = jnp.exp2(a * jnp.float32(-1.4426950408889634))
    p = jnp.full(e.shape, _LOG1P_C[0], jnp.float32)
    for c in _LOG1P_C[1:]:
        p = p * e + c
    ce = p - jnp.minimum(md, 0.0)
    sum_ref[...] += w2 * ce
    cnt_ref[...] += w2


def _rpn_loss_tc(match2d, logits2d):
    return pl.pallas_call(
        _tc_body,
        grid=(TC_STEPS,),
        in_specs=[
            pl.BlockSpec((TCG, 128), lambda i: (TC_BLK0 + i, 0)),
            pl.BlockSpec((2 * TCG, 128), lambda i: (TC_BLK0 + i, 0)),
        ],
        out_specs=[
            pl.BlockSpec((TCG, 128), lambda i: (0, 0)),
            pl.BlockSpec((TCG, 128), lambda i: (0, 0)),
        ],
        out_shape=[jax.ShapeDtypeStruct((TCG, 128), jnp.float32)] * 2,
    )(match2d, logits2d)


def kernel(rpn_match, rpn_class_logits):
    m_flat = rpn_match.reshape(TOTAL)
    lg_flat = (rpn_class_logits
               .reshape(B, A // GROUP, GROUP, 2)
               .transpose(0, 1, 3, 2)
               .reshape(TOTAL * 2))
    sc_sum, sc_cnt = _rpn_loss_sc(m_flat, lg_flat)
    tc_sum, tc_cnt = _rpn_loss_tc(m_flat.reshape(NGROUPS, GROUP),
                                  lg_flat.reshape(2 * NGROUPS, GROUP))
    s = jnp.sum(sc_sum) + jnp.sum(tc_sum)
    c = jnp.sum(sc_cnt) + jnp.sum(tc_cnt)
    return jnp.where(c > 0, s / jnp.maximum(c, 1.0), jnp.float32(0.0))


# speedup vs baseline: 1.1478x; 1.0139x over previous
import functools

import jax
import jax.numpy as jnp
from jax import lax
from jax.experimental import pallas as pl
from jax.experimental.pallas import tpu as pltpu
from jax.experimental.pallas import tpu_sc as plsc

NC = 2
NS = 16
L = 16
NW = NC * NS

B = 16
A = 262144
TOTAL = B * A
GROUP = 128
NGROUPS = TOTAL // GROUP

SC_FRAC16 = 10
SC_ANCHORS = TOTAL * SC_FRAC16 // 16
SC_GROUPS = SC_ANCHORS // GROUP

PER_W = SC_ANCHORS // NW
CHUNK = 8192
NCHUNK = PER_W // CHUNK
STEPS = CHUNK // GROUP
UNROLL = GROUP // L

_LOG1P_C = (
    -0.05486231128935009,
    0.2164085836818178,
    -0.46407070110262433,
    0.9954266617754363,
    0.00014158017492720682,
)


def _sc_body(match_hbm, logits_hbm, out_sum, out_cnt,
             mb0, mb1, lb0, lb1, acc_s, cnt_s, sm0, sm1, sl0, sl1):
    cid = lax.axis_index("c")
    sid = lax.axis_index("s")
    wid = sid * NC + cid
    mbase = wid * PER_W

    mbufs = (mb0, mb1)
    lbufs = (lb0, lb1)
    msems = (sm0, sm1)
    lsems = (sl0, sl1)

    def start(k):
        slot = k % 2
        off = mbase + k * CHUNK
        cm = pltpu.async_copy(match_hbm.at[pl.ds(off, CHUNK)],
                              mbufs[slot], msems[slot])
        cl = pltpu.async_copy(logits_hbm.at[pl.ds(2 * off, 2 * CHUNK)],
                              lbufs[slot], lsems[slot])
        return cm, cl

    pending = start(0)
    acc = jnp.zeros((L,), jnp.float32)
    cnt = jnp.zeros((L,), jnp.float32)
    for k in range(NCHUNK):
        nxt = start(k + 1) if k + 1 < NCHUNK else None
        pending[0].wait()
        pending[1].wait()
        mb = mbufs[k % 2]
        lb = lbufs[k % 2]

        def body(j, carry, mb=mb, lb=lb):
            acc, cnt = carry
            mo = j * GROUP
            lo = j * (2 * GROUP)
            for u in range(UNROLL):
                m = mb[pl.ds(mo + u * L, L)]
                l0 = lb[pl.ds(lo + u * L, L)]
                l1 = lb[pl.ds(lo + GROUP + u * L, L)]
                d = l1 - l0
                md = m * d
                w2 = m * m
                a = jnp.abs(d)
                e = jnp.exp(-a)
                p = jnp.full((L,), _LOG1P_C[0], jnp.float32)
                for c in _LOG1P_C[1:]:
                    p = p * e + c
                ce = p - jnp.minimum(md, 0.0)
                acc = acc + w2 * ce
                cnt = cnt + w2
            return acc, cnt

        acc, cnt = lax.fori_loop(0, STEPS, body, (acc, cnt))
        pending = nxt

    acc_s[...] = acc
    cnt_s[...] = cnt
    pltpu.sync_copy(acc_s, out_sum.at[wid])
    pltpu.sync_copy(cnt_s, out_cnt.at[wid])


_rpn_loss_sc = functools.partial(
    pl.kernel,
    out_type=(jax.ShapeDtypeStruct((NW, L), jnp.float32),
              jax.ShapeDtypeStruct((NW, L), jnp.float32)),
    mesh=plsc.VectorSubcoreMesh(core_axis_name="c", subcore_axis_name="s",
                                num_cores=NC, num_subcores=NS),
    compiler_params=pltpu.CompilerParams(needs_layout_passes=False),
    scratch_types=[
        pltpu.VMEM((CHUNK,), jnp.float32),
        pltpu.VMEM((CHUNK,), jnp.float32),
        pltpu.VMEM((2 * CHUNK,), jnp.float32),
        pltpu.VMEM((2 * CHUNK,), jnp.float32),
        pltpu.VMEM((L,), jnp.float32),
        pltpu.VMEM((L,), jnp.float32),
        pltpu.SemaphoreType.DMA,
        pltpu.SemaphoreType.DMA,
        pltpu.SemaphoreType.DMA,
        pltpu.SemaphoreType.DMA,
    ],
)(_sc_body)


TCG = 1024
TC_GROUPS = NGROUPS - SC_GROUPS
TC_STEPS = TC_GROUPS // TCG
TC_BLK0 = SC_GROUPS // TCG


def _tc_body(m_ref, x_ref, sum_ref, cnt_ref):
    i = pl.program_id(0)

    @pl.when(i == 0)
    def _init():
        sum_ref[...] = jnp.zeros_like(sum_ref)
        cnt_ref[...] = jnp.zeros_like(cnt_ref)

    m = m_ref[...]
    x = x_ref[...]
    x4 = x.reshape(TCG, 2, 128)
    xt = jnp.transpose(x4, (1, 0, 2))
    l0 = xt[0]
    l1 = xt[1]
    d = l1 - l0
    md = m * d
    w2 = m * m
    a = jnp.abs(d)
    e = jnp.exp2(a * jnp.float32(-1.4426950408889634))
    p = jnp.full(e.shape, _LOG1P_C[0], jnp.float32)
    for c in _LOG1P_C[1:]:
        p = p * e + c
    ce = p - jnp.minimum(md, 0.0)
    sum_ref[...] += w2 * ce
    cnt_ref[...] += w2


def _rpn_loss_tc(match2d, logits2d):
    return pl.pallas_call(
        _tc_body,
        grid=(TC_STEPS,),
        in_specs=[
            pl.BlockSpec((TCG, 128), lambda i: (TC_BLK0 + i, 0)),
            pl.BlockSpec((2 * TCG, 128), lambda i: (TC_BLK0 + i, 0)),
        ],
        out_specs=[
            pl.BlockSpec((TCG, 128), lambda i: (0, 0)),
            pl.BlockSpec((TCG, 128), lambda i: (0, 0)),
        ],
        out_shape=[jax.ShapeDtypeStruct((TCG, 128), jnp.float32)] * 2,
    )(match2d, logits2d)


def kernel(rpn_match, rpn_class_logits):
    m_flat = rpn_match.reshape(TOTAL)
    lg_flat = (rpn_class_logits
               .reshape(B, A // GROUP, GROUP, 2)
               .transpose(0, 1, 3, 2)
               .reshape(TOTAL * 2))
    sc_sum, sc_cnt = _rpn_loss_sc(m_flat, lg_flat)
    tc_sum, tc_cnt = _rpn_loss_tc(m_flat.reshape(NGROUPS, GROUP),
                                  lg_flat.reshape(2 * NGROUPS, GROUP))
    s = jnp.sum(sc_sum) + jnp.sum(tc_sum)
    c = jnp.sum(sc_cnt) + jnp.sum(tc_cnt)
    return jnp.where(c > 0, s / jnp.maximum(c, 1.0), jnp.float32(0.0))
